# Initial kernel scaffold; baseline (speedup 1.0000x reference)
import jax
import jax.numpy as jnp
from jax.experimental import pallas as pl


def kernel(x, edge_index, W, att_src, att_dst, bias):
    raise NotImplementedError("write your pallas kernel here")


# speedup vs baseline: 61.7065x; 61.7065x over previous
import functools

import jax
import jax.numpy as jnp
from jax import lax
from jax.experimental import pallas as pl
from jax.experimental.pallas import tpu as pltpu
from jax.experimental.pallas import tpu_sc as plsc

N = 50000
E = 800000
D = 32
H = 4
C = 64
HC = H * C

NPART = 6
NP = 8512
NPAD = NPART * NP
SPROWS = NP + 8
EPAD = 819200
ROWS_PER_TILE = 400
CHUNK_ROWS = 16
NCHUNK = ROWS_PER_TILE // CHUNK_ROWS
SHARE = NP // 16
BN = 1064
GRID = NPAD // BN


def _pre_body(x_ref, w_ref, asf_ref, adf_ref, as_ref, ad_ref):
    xb = x_ref[...]
    hb = jnp.dot(xb, w_ref[...], preferred_element_type=jnp.float32)
    ps = hb * asf_ref[...]
    pd = hb * adf_ref[...]
    as_ref[...] = jnp.concatenate(
        [jnp.sum(ps[:, h * C:(h + 1) * C], axis=1, keepdims=True) for h in range(H)],
        axis=1)
    ad_ref[...] = jnp.concatenate(
        [jnp.sum(pd[:, h * C:(h + 1) * C], axis=1, keepdims=True) for h in range(H)],
        axis=1)


def _pre(xpad, W, asf, adf):
    return pl.pallas_call(
        _pre_body,
        grid=(GRID,),
        in_specs=[
            pl.BlockSpec((BN, D), lambda i: (i, 0)),
            pl.BlockSpec((D, HC), lambda i: (0, 0)),
            pl.BlockSpec((1, HC), lambda i: (0, 0)),
            pl.BlockSpec((1, HC), lambda i: (0, 0)),
        ],
        out_specs=[
            pl.BlockSpec((BN, H), lambda i: (i, 0)),
            pl.BlockSpec((BN, H), lambda i: (i, 0)),
        ],
        out_shape=[
            jax.ShapeDtypeStruct((NPAD, H), jnp.float32),
            jax.ShapeDtypeStruct((NPAD, H), jnp.float32),
        ],
    )(xpad, W, asf, adf)


def _post_body(acc_ref, den_ref, w_ref, b_ref, o_ref):
    accb = acc_ref[...]
    denb = den_ref[...]
    outs = []
    for h in range(H):
        a = accb[:, h * D:(h + 1) * D] / (denb[:, h:h + 1] + 1e-16)
        outs.append(jnp.dot(a, w_ref[:, h * C:(h + 1) * C],
                            preferred_element_type=jnp.float32))
    o_ref[...] = jnp.concatenate(outs, axis=1) + b_ref[...]


def _post(acc, den, W, b2d):
    return pl.pallas_call(
        _post_body,
        grid=(GRID,),
        in_specs=[
            pl.BlockSpec((BN, H * D), lambda i: (i, 0)),
            pl.BlockSpec((BN, 16), lambda i: (i, 0)),
            pl.BlockSpec((D, HC), lambda i: (0, 0)),
            pl.BlockSpec((1, HC), lambda i: (0, 0)),
        ],
        out_specs=pl.BlockSpec((BN, HC), lambda i: (i, 0)),
        out_shape=jax.ShapeDtypeStruct((NPAD, HC), jnp.float32),
    )(acc, den, W, b2d)


def _sc_body(src_hbm, dst_hbm, x_hbm, as_hbm, ad_hbm, m_hbm, zz_hbm, zd_hbm,
             acc_hbm, den_hbm,
             acc_sp, den_sp, srcc, dstc, qs, qd, stgs, stgd, stgl,
             xb, asb, adb, exb, msgb, mb,
             sem0, sem1, sem2):
    cix = lax.axis_index("c")
    six = lax.axis_index("s")
    it16 = lax.iota(jnp.int32, 16)

    pltpu.sync_copy(m_hbm, mb)

    zi = jnp.zeros((16,), jnp.int32)

    def zq(i, carry):
        qs[pl.ds(i * 16, 16)] = zi
        qd[pl.ds(i * 16, 16)] = zi
        return carry

    lax.fori_loop(0, 128, zq, 0)

    tbase = six * ROWS_PER_TILE
    off0 = six * SHARE

    for p in range(NPART // 2):
        part = 2 * p + cix
        base = part * NP

        pltpu.sync_copy(zz_hbm, acc_sp.at[pl.ds(off0, SHARE)])
        pltpu.sync_copy(zd_hbm, den_sp.at[pl.ds(off0, SHARE)])
        plsc.subcore_barrier()

        def chunk_body(ch, carry):
            r0 = tbase + ch * CHUNK_ROWS
            pltpu.sync_copy(src_hbm.at[pl.ds(r0, CHUNK_ROWS)], srcc)
            pltpu.sync_copy(dst_hbm.at[pl.ds(r0, CHUNK_ROWS)], dstc)

            def scan_body(g, cnt):
                r = lax.shift_right_logical(g, 3)
                col = (g & 7) * 16
                vd = dstc[r, pl.ds(col, 16)]
                vs = srcc[r, pl.ds(col, 16)]
                loc = vd - base
                msk = (loc >= 0) & (loc < NP)
                pos = plsc.cumsum(msk.astype(jnp.int32))
                qi = cnt + pos - 1
                plsc.store_scatter(qs, [qi], vs, mask=msk)
                plsc.store_scatter(qd, [qi], vd, mask=msk)
                return cnt + pos[15]

            cnt = lax.fori_loop(0, CHUNK_ROWS * 8, scan_body, jnp.int32(0))
            nb = lax.shift_right_logical(cnt + 127, 7)

            def batch_body(j, carry2):
                for k in range(8):
                    off = j * 128 + k * 16
                    vs = qs[pl.ds(off, 16)]
                    vd = qd[pl.ds(off, 16)]
                    valid = (off + it16) < cnt
                    stgs[0, pl.ds(k * 16, 16)] = vs
                    stgd[0, pl.ds(k * 16, 16)] = vd
                    stgl[0, pl.ds(k * 16, 16)] = jnp.where(valid, vd - base, NP)
                ca = pltpu.async_copy(as_hbm.at[stgs.at[0]], asb, sem0)
                cb = pltpu.async_copy(ad_hbm.at[stgd.at[0]], adb, sem1)
                cx = pltpu.async_copy(x_hbm.at[stgs.at[0]], xb, sem2)
                ca.wait()
                cb.wait()
                cx.wait()
                mv = mb[...]
                for k in range(8):
                    e_vec = it16 + k * 16
                    valid = (j * 128 + k * 16 + it16) < cnt
                    vmf = jnp.where(valid, 1.0, 0.0)
                    for h in range(H):
                        hv = jnp.full((16,), h, jnp.int32)
                        av = plsc.load_gather(asb, [e_vec, hv])
                        dv = plsc.load_gather(adb, [e_vec, hv])
                        ev = av + dv
                        ev = jnp.where(ev >= 0, ev, 0.2 * ev)
                        exv = jnp.exp(ev - mv[h]) * vmf
                        plsc.store_scatter(exb, [e_vec, hv], exv)
                pltpu.sync_copy(exb, den_sp.at[stgl.at[0]], add=True)

                def mbody(e, carry3):
                    x0 = xb[e, pl.ds(0, 16)]
                    x1 = xb[e, pl.ds(16, 16)]
                    exrow = exb[e, pl.ds(0, 16)]
                    for h in range(H):
                        sc = exrow[h]
                        msgb[e, pl.ds(h * 32, 16)] = sc * x0
                        msgb[e, pl.ds(h * 32 + 16, 16)] = sc * x1
                    return carry3

                lax.fori_loop(0, 128, mbody, 0)
                pltpu.sync_copy(msgb, acc_sp.at[stgl.at[0]], add=True)
                return carry2

            lax.fori_loop(0, nb, batch_body, 0)
            return carry

        lax.fori_loop(0, NCHUNK, chunk_body, 0)
        plsc.subcore_barrier()

        pltpu.sync_copy(acc_sp.at[pl.ds(off0, SHARE)],
                        acc_hbm.at[pl.ds(base + off0, SHARE)])
        pltpu.sync_copy(den_sp.at[pl.ds(off0, SHARE)],
                        den_hbm.at[pl.ds(base + off0, SHARE)])


def _sc_edge(src2d, dst2d, xpad, a_s, a_d, m16, zz, zd):
    mesh = plsc.VectorSubcoreMesh(core_axis_name="c", subcore_axis_name="s")
    f32 = jnp.float32
    i32 = jnp.int32
    fn = functools.partial(
        pl.kernel,
        mesh=mesh,
        compiler_params=pltpu.CompilerParams(
            use_tc_tiling_on_sc=False, needs_layout_passes=False),
        out_type=(
            jax.ShapeDtypeStruct((NPAD, H * D), f32),
            jax.ShapeDtypeStruct((NPAD, 16), f32),
        ),
        scratch_types=[
            pltpu.VMEM_SHARED((SPROWS, H * D), f32),
            pltpu.VMEM_SHARED((SPROWS, 16), f32),
            pltpu.VMEM((CHUNK_ROWS, 128), i32),
            pltpu.VMEM((CHUNK_ROWS, 128), i32),
            pltpu.VMEM((2048,), i32),
            pltpu.VMEM((2048,), i32),
            pltpu.VMEM((1, 128), i32),
            pltpu.VMEM((1, 128), i32),
            pltpu.VMEM((1, 128), i32),
            pltpu.VMEM((128, D), f32),
            pltpu.VMEM((128, 16), f32),
            pltpu.VMEM((128, 16), f32),
            pltpu.VMEM((128, 16), f32),
            pltpu.VMEM((128, H * D), f32),
            pltpu.VMEM((16,), f32),
            pltpu.SemaphoreType.DMA,
            pltpu.SemaphoreType.DMA,
            pltpu.SemaphoreType.DMA,
        ],
    )(_sc_body)
    return fn(src2d, dst2d, xpad, a_s, a_d, m16, zz, zd)


@jax.jit
def kernel(x, edge_index, W, att_src, att_dst, bias):
    f32 = jnp.float32
    src = edge_index[0]
    dst = edge_index[1]
    xpad = jnp.pad(x, ((0, NPAD - N), (0, 0)))
    asf = att_src.reshape(1, HC)
    adf = att_dst.reshape(1, HC)
    a_s, a_d = _pre(xpad, W, asf, adf)
    m4 = jnp.maximum(jnp.max(a_s, axis=0) + jnp.max(a_d, axis=0), 0.0)
    m16 = jnp.pad(m4, (0, 12))
    a_s16 = jnp.pad(a_s, ((0, 0), (0, 12)))
    a_d16 = jnp.pad(a_d, ((0, 0), (0, 12)))
    src2d = jnp.pad(src, (0, EPAD - E)).reshape(EPAD // 128, 128)
    dst2d = jnp.pad(dst, (0, EPAD - E),
                    constant_values=NPAD).reshape(EPAD // 128, 128)
    zz = jnp.zeros((SHARE, 128), f32)
    zd = jnp.zeros((SHARE, 16), f32)
    acc, den = _sc_edge(src2d, dst2d, xpad, a_s16, a_d16, m16, zz, zd)
    out = _post(acc, den, W, bias.reshape(1, HC))
    return out[:N]


# speedup vs baseline: 65.6646x; 1.0641x over previous
import functools

import jax
import jax.numpy as jnp
from jax import lax
from jax.experimental import pallas as pl
from jax.experimental.pallas import tpu as pltpu
from jax.experimental.pallas import tpu_sc as plsc

N = 50000
E = 800000
D = 32
H = 4
C = 64
HC = H * C

NPART = 6
NP = 8512
NPAD = NPART * NP
SPROWS = NP + 8
EPAD = 819200
ROWS_PER_TILE = 400
CHUNK_ROWS = 16
NCHUNK = ROWS_PER_TILE // CHUNK_ROWS
SHARE = NP // 16
BN = 1064
GRID = NPAD // BN


def _pre_body(x_ref, w_ref, asf_ref, adf_ref, as_ref, ad_ref):
    xb = x_ref[...]
    hb = jnp.dot(xb, w_ref[...], preferred_element_type=jnp.float32)
    ps = hb * asf_ref[...]
    pd = hb * adf_ref[...]
    as_ref[...] = jnp.concatenate(
        [jnp.sum(ps[:, h * C:(h + 1) * C], axis=1, keepdims=True) for h in range(H)],
        axis=1)
    ad_ref[...] = jnp.concatenate(
        [jnp.sum(pd[:, h * C:(h + 1) * C], axis=1, keepdims=True) for h in range(H)],
        axis=1)


def _pre(xpad, W, asf, adf):
    return pl.pallas_call(
        _pre_body,
        grid=(GRID,),
        in_specs=[
            pl.BlockSpec((BN, D), lambda i: (i, 0)),
            pl.BlockSpec((D, HC), lambda i: (0, 0)),
            pl.BlockSpec((1, HC), lambda i: (0, 0)),
            pl.BlockSpec((1, HC), lambda i: (0, 0)),
        ],
        out_specs=[
            pl.BlockSpec((BN, H), lambda i: (i, 0)),
            pl.BlockSpec((BN, H), lambda i: (i, 0)),
        ],
        out_shape=[
            jax.ShapeDtypeStruct((NPAD, H), jnp.float32),
            jax.ShapeDtypeStruct((NPAD, H), jnp.float32),
        ],
    )(xpad, W, asf, adf)


def _post_body(acc_ref, den_ref, w_ref, b_ref, o_ref):
    accb = acc_ref[...]
    denb = den_ref[...]
    outs = []
    for h in range(H):
        a = accb[:, h * D:(h + 1) * D] / (denb[:, h:h + 1] + 1e-16)
        outs.append(jnp.dot(a, w_ref[:, h * C:(h + 1) * C],
                            preferred_element_type=jnp.float32))
    o_ref[...] = jnp.concatenate(outs, axis=1) + b_ref[...]


def _post(acc, den, W, b2d):
    return pl.pallas_call(
        _post_body,
        grid=(GRID,),
        in_specs=[
            pl.BlockSpec((BN, H * D), lambda i: (i, 0)),
            pl.BlockSpec((BN, 16), lambda i: (i, 0)),
            pl.BlockSpec((D, HC), lambda i: (0, 0)),
            pl.BlockSpec((1, HC), lambda i: (0, 0)),
        ],
        out_specs=pl.BlockSpec((BN, HC), lambda i: (i, 0)),
        out_shape=jax.ShapeDtypeStruct((NPAD, HC), jnp.float32),
    )(acc, den, W, b2d)


def _sc_body(src_hbm, dst_hbm, x_hbm, ad_hbm, m_hbm, zz_hbm, zd_hbm,
             acc_hbm, den_hbm,
             acc_sp, den_sp, srcc, dstc, qs, qd, stgs, stgd, stgl,
             xb, adb, exb, msgb, mb,
             sem0, sem1, sem2):
    cix = lax.axis_index("c")
    six = lax.axis_index("s")
    it16 = lax.iota(jnp.int32, 16)

    pltpu.sync_copy(m_hbm, mb)

    zi = jnp.zeros((16,), jnp.int32)

    def zq(i, carry):
        qs[pl.ds(i * 16, 16)] = zi
        qd[pl.ds(i * 16, 16)] = zi
        return carry

    lax.fori_loop(0, 128, zq, 0)

    tbase = six * ROWS_PER_TILE
    off0 = six * SHARE

    for p in range(NPART // 2):
        part = 2 * p + cix
        base = part * NP

        pltpu.sync_copy(zz_hbm, acc_sp.at[pl.ds(off0, SHARE)])
        pltpu.sync_copy(zd_hbm, den_sp.at[pl.ds(off0, SHARE)])
        plsc.subcore_barrier()

        def chunk_body(ch, carry):
            r0 = tbase + ch * CHUNK_ROWS
            pltpu.sync_copy(src_hbm.at[pl.ds(r0, CHUNK_ROWS)], srcc)
            pltpu.sync_copy(dst_hbm.at[pl.ds(r0, CHUNK_ROWS)], dstc)

            def scan_body(g, cnt):
                r = lax.shift_right_logical(g, 3)
                col = (g & 7) * 16
                vd = dstc[r, pl.ds(col, 16)]
                vs = srcc[r, pl.ds(col, 16)]
                loc = vd - base
                msk = (loc >= 0) & (loc < NP)
                pos = plsc.cumsum(msk.astype(jnp.int32))
                qi = cnt + pos - 1
                plsc.store_scatter(qs, [qi], vs, mask=msk)
                plsc.store_scatter(qd, [qi], vd, mask=msk)
                return cnt + pos[15]

            cnt = lax.fori_loop(0, CHUNK_ROWS * 8, scan_body, jnp.int32(0))
            nb = lax.shift_right_logical(cnt + 127, 7)

            def batch_body(j, carry2):
                for k in range(8):
                    off = j * 128 + k * 16
                    vs = qs[pl.ds(off, 16)]
                    vd = qd[pl.ds(off, 16)]
                    valid = (off + it16) < cnt
                    stgs[0, pl.ds(k * 16, 16)] = vs
                    stgd[0, pl.ds(k * 16, 16)] = vd
                    stgl[0, pl.ds(k * 16, 16)] = jnp.where(valid, vd - base, NP)
                cb = pltpu.async_copy(ad_hbm.at[stgd.at[0]], adb, sem1)
                cx = pltpu.async_copy(x_hbm.at[stgs.at[0]], xb, sem2)
                cb.wait()
                cx.wait()
                mv = mb[...]
                for k in range(8):
                    e_vec = it16 + k * 16
                    valid = (j * 128 + k * 16 + it16) < cnt
                    vmf = jnp.where(valid, 1.0, 0.0)
                    for h in range(H):
                        hv = jnp.full((16,), h, jnp.int32)
                        av = plsc.load_gather(xb, [e_vec, hv + D])
                        dv = plsc.load_gather(adb, [e_vec, hv])
                        ev = av + dv
                        ev = jnp.where(ev >= 0, ev, 0.2 * ev)
                        exv = jnp.exp(ev - mv[h]) * vmf
                        plsc.store_scatter(exb, [e_vec, hv], exv)
                pltpu.sync_copy(exb, den_sp.at[stgl.at[0]], add=True)

                @plsc.parallel_loop(0, 128, unroll=4)
                def _msg(e):
                    x0 = xb[e, pl.ds(0, 16)]
                    x1 = xb[e, pl.ds(16, 16)]
                    exrow = exb[e, pl.ds(0, 16)]
                    for h in range(H):
                        sc = exrow[h]
                        msgb[e, pl.ds(h * 32, 16)] = sc * x0
                        msgb[e, pl.ds(h * 32 + 16, 16)] = sc * x1
                pltpu.sync_copy(msgb, acc_sp.at[stgl.at[0]], add=True)
                return carry2

            lax.fori_loop(0, nb, batch_body, 0)
            return carry

        lax.fori_loop(0, NCHUNK, chunk_body, 0)
        plsc.subcore_barrier()

        pltpu.sync_copy(acc_sp.at[pl.ds(off0, SHARE)],
                        acc_hbm.at[pl.ds(base + off0, SHARE)])
        pltpu.sync_copy(den_sp.at[pl.ds(off0, SHARE)],
                        den_hbm.at[pl.ds(base + off0, SHARE)])


def _sc_edge(src2d, dst2d, xt, a_d, m16, zz, zd):
    mesh = plsc.VectorSubcoreMesh(core_axis_name="c", subcore_axis_name="s")
    f32 = jnp.float32
    i32 = jnp.int32
    fn = functools.partial(
        pl.kernel,
        mesh=mesh,
        compiler_params=pltpu.CompilerParams(
            use_tc_tiling_on_sc=False, needs_layout_passes=False),
        out_type=(
            jax.ShapeDtypeStruct((NPAD, H * D), f32),
            jax.ShapeDtypeStruct((NPAD, 16), f32),
        ),
        scratch_types=[
            pltpu.VMEM_SHARED((SPROWS, H * D), f32),
            pltpu.VMEM_SHARED((SPROWS, 16), f32),
            pltpu.VMEM((CHUNK_ROWS, 128), i32),
            pltpu.VMEM((CHUNK_ROWS, 128), i32),
            pltpu.VMEM((2048,), i32),
            pltpu.VMEM((2048,), i32),
            pltpu.VMEM((1, 128), i32),
            pltpu.VMEM((1, 128), i32),
            pltpu.VMEM((1, 128), i32),
            pltpu.VMEM((128, D + 16), f32),
            pltpu.VMEM((128, 16), f32),
            pltpu.VMEM((128, 16), f32),
            pltpu.VMEM((128, H * D), f32),
            pltpu.VMEM((16,), f32),
            pltpu.SemaphoreType.DMA,
            pltpu.SemaphoreType.DMA,
            pltpu.SemaphoreType.DMA,
        ],
    )(_sc_body)
    return fn(src2d, dst2d, xt, a_d, m16, zz, zd)


@jax.jit
def kernel(x, edge_index, W, att_src, att_dst, bias):
    f32 = jnp.float32
    src = edge_index[0]
    dst = edge_index[1]
    xpad = jnp.pad(x, ((0, NPAD - N), (0, 0)))
    asf = att_src.reshape(1, HC)
    adf = att_dst.reshape(1, HC)
    a_s, a_d = _pre(xpad, W, asf, adf)
    m4 = jnp.maximum(jnp.max(a_s, axis=0) + jnp.max(a_d, axis=0), 0.0)
    m16 = jnp.pad(m4, (0, 12))
    xt = jnp.concatenate(
        [xpad, a_s, jnp.zeros((NPAD, 12), f32)], axis=1)
    a_d16 = jnp.pad(a_d, ((0, 0), (0, 12)))
    src2d = jnp.pad(src, (0, EPAD - E)).reshape(EPAD // 128, 128)
    dst2d = jnp.pad(dst, (0, EPAD - E),
                    constant_values=NPAD).reshape(EPAD // 128, 128)
    zz = jnp.zeros((SHARE, 128), f32)
    zd = jnp.zeros((SHARE, 16), f32)
    acc, den = _sc_edge(src2d, dst2d, xt, a_d16, m16, zz, zd)
    out = _post(acc, den, W, bias.reshape(1, HC))
    return out[:N]
